# baseline (speedup 1.0000x reference)
import functools

import jax
import jax.numpy as jnp
from jax import lax
from jax.experimental import pallas as pl
from jax.experimental.pallas import tpu as pltpu
from jax.experimental.pallas import tpu_sc as plsc

_N = 10000
_E = 320000
_D = 128
_NC, _NS = 2, 16
_NW = _NC * _NS
_EPW = _E // _NW
_CHUNK = 80
_NCHUNK = _EPW // _CHUNK
_NPAD = 10240
_ROWS_T = _NPAD // _NS

_BLK = 1000
_GRID = _N // _BLK


def _mesh():
    return plsc.VectorSubcoreMesh(
        core_axis_name="c", subcore_axis_name="s",
        num_cores=_NC, num_subcores=_NS)


_DHR = 80


@functools.cache
def _get_deg_kernel():
    return functools.partial(
        pl.kernel,
        out_type=jax.ShapeDtypeStruct((_NC, _DHR, _D), jnp.float32),
        mesh=_mesh(),
        scratch_types=[
            pltpu.VMEM((_NCHUNK, _CHUNK), jnp.int32),
            pltpu.VMEM((_DHR, _D), jnp.float32),
            pltpu.VMEM((_DHR,), jnp.int32),
            pltpu.VMEM_SHARED((_DHR, _D), jnp.float32),
        ],
        compiler_params=pltpu.CompilerParams(needs_layout_passes=False),
    )(_deg_body)


def _deg_body(pk_hbm, zeros_hbm, out_hbm, pk_v, hist, rowidx, accum):
    c = lax.axis_index("c")
    s = lax.axis_index("s")
    w = s * _NC + c
    pltpu.sync_copy(pk_hbm.at[w], pk_v)

    zeros16 = jnp.zeros((16,), jnp.float32)

    def zbody(i, carry):
        hist[i >> 3, pl.ds((i & 7) * 16, 16)] = zeros16
        return carry
    lax.fori_loop(0, _DHR * 8, zbody, 0)

    iota16 = lax.broadcasted_iota(jnp.int32, (16,), 0)
    for i in range(_DHR // 16):
        rowidx[pl.ds(i * 16, 16)] = iota16 + i * 16

    ones = jnp.ones((16,), jnp.float32)

    def cbody(i, carry):
        idx = pk_v[i // 5, pl.ds((i % 5) * 16, 16)] >> 16
        plsc.addupdate_scatter(hist, [idx >> 7, idx & 127], ones)
        return carry
    lax.fori_loop(0, _EPW // 16, cbody, 0)

    @pl.when(s < 10)
    def _():
        pltpu.sync_copy(zeros_hbm.at[pl.ds(8 * s, 8)],
                        accum.at[pl.ds(8 * s, 8)])
    plsc.subcore_barrier()
    pltpu.sync_copy(hist, accum.at[rowidx], add=True)
    plsc.subcore_barrier()

    @pl.when(s < 10)
    def _():
        pltpu.sync_copy(accum.at[pl.ds(8 * s, 8)],
                        out_hbm.at[c, pl.ds(8 * s, 8)])


_NBUF = 3


@functools.cache
def _get_mp_kernel():
    return functools.partial(
        pl.kernel,
        out_type=jax.ShapeDtypeStruct((_NC, _NPAD, _D), jnp.float32),
        mesh=_mesh(),
        scratch_types=[
            pltpu.VMEM_SHARED((_NPAD, _D), jnp.float32),
        ]
        + [pltpu.VMEM((_CHUNK, _D), jnp.float32) for _ in range(_NBUF)]
        + [pltpu.VMEM((_CHUNK,), jnp.int32) for _ in range(3 * _NBUF)]
        + [pltpu.SemaphoreType.DMA for _ in range(3 * _NBUF)],
        compiler_params=pltpu.CompilerParams(needs_layout_passes=False),
    )(_mp_body)


def _mp_body(g_hbm, pk_hbm, zeros_hbm, out_hbm,
             accum, *bufs_and_sems):
    bufs = bufs_and_sems[:_NBUF]
    pkbufs = bufs_and_sems[_NBUF:2 * _NBUF]
    srcbufs = bufs_and_sems[2 * _NBUF:3 * _NBUF]
    dstbufs = bufs_and_sems[3 * _NBUF:4 * _NBUF]
    gsems = bufs_and_sems[4 * _NBUF:5 * _NBUF]
    ssems = bufs_and_sems[5 * _NBUF:6 * _NBUF]
    psems = bufs_and_sems[6 * _NBUF:]
    c = lax.axis_index("c")
    s = lax.axis_index("s")
    w = s * _NC + c
    pltpu.sync_copy(zeros_hbm.at[pl.ds(s * _ROWS_T, _ROWS_T)],
                    accum.at[pl.ds(s * _ROWS_T, _ROWS_T)])

    def pkstart(j, b):
        pltpu.async_copy(pk_hbm.at[w, j], pkbufs[b], psems[b])

    def pkwait(b):
        pltpu.make_async_copy(pk_hbm.at[w, 0], pkbufs[b], psems[b]).wait()

    def unpack(b):
        for i in range(_CHUNK // 16):
            p = pkbufs[b][pl.ds(i * 16, 16)]
            srcbufs[b][pl.ds(i * 16, 16)] = p & 0xFFFF
            dstbufs[b][pl.ds(i * 16, 16)] = p >> 16

    def gstart(b):
        pltpu.async_copy(g_hbm.at[srcbufs[b]], bufs[b], gsems[b])

    def gwait(b):
        pltpu.make_async_copy(g_hbm.at[srcbufs[b]], bufs[b],
                              gsems[b]).wait()

    def sstart(b):
        pltpu.async_copy(bufs[b], accum.at[dstbufs[b]], ssems[b], add=True)

    def swait(b):
        pltpu.make_async_copy(bufs[b], accum.at[dstbufs[b]],
                              ssems[b]).wait()

    for b in range(_NBUF):
        pkstart(b, b)
    plsc.subcore_barrier()
    for b in range(_NBUF):
        pkwait(b)
        unpack(b)
        gstart(b)
        pkstart(b + _NBUF, b)

    def body(k, carry):
        for b in range(_NBUF):
            gwait(b)
            sstart(b)
        for b in range(_NBUF):
            jn = k * _NBUF + b + _NBUF

            @pl.when(jn < _NCHUNK)
            def _():
                swait(b)
                pkwait(b)
                unpack(b)
                gstart(b)

                @pl.when(jn + _NBUF < _NCHUNK)
                def _():
                    pkstart(jn + _NBUF, b)
        return carry
    lax.fori_loop(0, _NCHUNK // _NBUF, body, 0)

    for b in range(_NCHUNK % _NBUF):
        gwait(b)
        sstart(b)
    for b in range(_NBUF):
        swait(b)

    plsc.subcore_barrier()
    pltpu.sync_copy(accum.at[pl.ds(s * _ROWS_T, _ROWS_T)],
                    out_hbm.at[c, pl.ds(s * _ROWS_T, _ROWS_T)])


def _elu(x):
    return jnp.where(x > 0, x, jnp.exp(jnp.where(x > 0, 0.0, x)) - 1.0)


def _row_spec(width):
    return pl.BlockSpec((_BLK, width), lambda i: (i, 0))


def _full_spec(shape):
    return pl.BlockSpec(shape, lambda i: tuple(0 for _ in shape))


def _t_spec():
    return pl.BlockSpec((_NC, _BLK, _D), lambda i: (0, i, 0))


def _tc0_body(x_ref, w0_ref, d0_ref, d1_ref, qe_ref, f0w_ref, f0b_ref,
              g0_ref, dinv_ref, q_ref):
    dinv = lax.rsqrt(d0_ref[...] + d1_ref[...] + 1.0)
    dinv_ref[...] = dinv
    g0_ref[...] = dinv * jnp.dot(x_ref[...], w0_ref[...],
                                 preferred_element_type=jnp.float32)

    @pl.when(pl.program_id(0) == 0)
    def _():
        q = jnp.dot(qe_ref[...], f0w_ref[...],
                    preferred_element_type=jnp.float32) + f0b_ref[...]
        q_ref[...] = _elu(q)


def _tc0(x, W0, d0, d1, qe, f0w, f0b):
    return pl.pallas_call(
        _tc0_body,
        grid=(_GRID,),
        in_specs=[_row_spec(_D), _full_spec((_D, _D)), _row_spec(1),
                  _row_spec(1), _full_spec((64, 256)), _full_spec((256, 64)),
                  _full_spec((1, 64))],
        out_specs=[_row_spec(_D), _row_spec(1), _full_spec((64, 64))],
        out_shape=[jax.ShapeDtypeStruct((_N, _D), jnp.float32),
                   jax.ShapeDtypeStruct((_N, 1), jnp.float32),
                   jax.ShapeDtypeStruct((64, 64), jnp.float32)],
    )(x, W0, d0, d1, qe, f0w, f0b)


def _tc_mid_body(emit_h, t_ref, g_ref, dinv_ref, b_ref, w_ref, *out_refs):
    dinv = dinv_ref[...]
    h = _elu(dinv * (t_ref[0] + t_ref[1] + g_ref[...]) + b_ref[...])
    if emit_h:
        out_refs[1][...] = h
    out_refs[0][...] = dinv * jnp.dot(h, w_ref[...],
                                      preferred_element_type=jnp.float32)


def _tc_mid(t, g, dinv, b, W_next, emit_h):
    out_specs = [_row_spec(_D)]
    out_shape = [jax.ShapeDtypeStruct((_N, _D), jnp.float32)]
    if emit_h:
        out_specs.append(_row_spec(_D))
        out_shape.append(jax.ShapeDtypeStruct((_N, _D), jnp.float32))
    return pl.pallas_call(
        functools.partial(_tc_mid_body, emit_h),
        grid=(_GRID,),
        in_specs=[_t_spec(), _row_spec(_D), _row_spec(1),
                  _full_spec((1, _D)), _full_spec((_D, _D))],
        out_specs=out_specs,
        out_shape=out_shape,
    )(t, g, dinv, b, W_next)


def _tc_final_body(t_ref, g_ref, dinv_ref, b_ref, r_ref, q_ref, batch_ref,
                   f1a_ref, f1b_ref, f1bias_ref, f2w_ref, f2b_ref, out_ref):
    dinv = dinv_ref[...]
    h = _elu(dinv * (t_ref[0] + t_ref[1] + g_ref[...]) + b_ref[...])
    h = h + r_ref[...]
    onehot = (batch_ref[...] ==
              lax.broadcasted_iota(jnp.int32, (_BLK, 64), 1)
              ).astype(jnp.float32)
    qx = jnp.dot(onehot, q_ref[...], preferred_element_type=jnp.float32)
    pre = (jnp.dot(h, f1a_ref[...], preferred_element_type=jnp.float32)
           + jnp.dot(qx, f1b_ref[...], preferred_element_type=jnp.float32)
           + f1bias_ref[...])
    h2 = _elu(pre)
    out_ref[...] = (jnp.dot(h2, f2w_ref[...],
                            preferred_element_type=jnp.float32)
                    + f2b_ref[...])


def _tc_final(t, g, dinv, b, r, q, batch2, f1a, f1b, f1bias, f2w, f2b):
    return pl.pallas_call(
        _tc_final_body,
        grid=(_GRID,),
        in_specs=[_t_spec(), _row_spec(_D), _row_spec(1),
                  _full_spec((1, _D)), _row_spec(_D), _full_spec((64, 64)),
                  _row_spec(1), _full_spec((_D, _D)), _full_spec((64, _D)),
                  _full_spec((1, _D)), _full_spec((_D, _D)),
                  _full_spec((1, _D))],
        out_specs=_row_spec(_D),
        out_shape=jax.ShapeDtypeStruct((_N, _D), jnp.float32),
    )(t, g, dinv, b, r, q, batch2, f1a, f1b, f1bias, f2w, f2b)


def kernel(x, edge_index, batch, question_embedding,
           W0, b0, W1, b1, W2, b2, W3, b3,
           fc0_W, fc0_b, fc1_W, fc1_b, fc2_W, fc2_b):
    src = edge_index[0].reshape(_NW, _NCHUNK, _CHUNK)
    dst = edge_index[1].reshape(_NW, _NCHUNK, _CHUNK)
    packed = src | (dst << 16)
    zeros_nd = jnp.zeros((_NPAD, _D), jnp.float32)

    degp = _get_deg_kernel()(packed, zeros_nd).reshape(_NC, _DHR * _D)
    d0 = degp[0, :_N].reshape(_N, 1)
    d1 = degp[1, :_N].reshape(_N, 1)

    g0, dinv, q = _tc0(x, W0, d0, d1, question_embedding, fc0_W,
                       fc0_b.reshape(1, 64))

    t = _get_mp_kernel()(g0, packed, zeros_nd)
    (g1,) = _tc_mid(t, g0, dinv, b0.reshape(1, _D), W1, False)
    t = _get_mp_kernel()(g1, packed, zeros_nd)
    g2, r = _tc_mid(t, g1, dinv, b1.reshape(1, _D), W2, True)
    t = _get_mp_kernel()(g2, packed, zeros_nd)
    (g3,) = _tc_mid(t, g2, dinv, b2.reshape(1, _D), W3, False)
    t = _get_mp_kernel()(g3, packed, zeros_nd)

    out = _tc_final(t, g3, dinv, b3.reshape(1, _D), r, q,
                    batch.reshape(_N, 1),
                    fc1_W[:_D], fc1_W[_D:], fc1_b.reshape(1, _D),
                    fc2_W, fc2_b.reshape(1, _D))
    return out


# speedup vs baseline: 1.0166x; 1.0166x over previous
import functools

import jax
import jax.numpy as jnp
from jax import lax
from jax.experimental import pallas as pl
from jax.experimental.pallas import tpu as pltpu
from jax.experimental.pallas import tpu_sc as plsc

_N = 10000
_E = 320000
_D = 128
_NC, _NS = 2, 16
_NW = _NC * _NS
_EPW = _E // _NW
_CHUNK = 80
_NCHUNK = _EPW // _CHUNK
_NPAD = 10240
_ROWS_T = _NPAD // _NS

_BLK = 1000
_GRID = _N // _BLK


def _mesh():
    return plsc.VectorSubcoreMesh(
        core_axis_name="c", subcore_axis_name="s",
        num_cores=_NC, num_subcores=_NS)


_DHR = 80


@functools.cache
def _get_deg_kernel():
    return functools.partial(
        pl.kernel,
        out_type=jax.ShapeDtypeStruct((_NC, _DHR, _D), jnp.float32),
        mesh=_mesh(),
        scratch_types=[
            pltpu.VMEM((_NCHUNK, _CHUNK), jnp.int32),
            pltpu.VMEM((_DHR, _D), jnp.float32),
            pltpu.VMEM((_DHR,), jnp.int32),
            pltpu.VMEM_SHARED((_DHR, _D), jnp.float32),
        ],
        compiler_params=pltpu.CompilerParams(needs_layout_passes=False),
    )(_deg_body)


def _deg_body(pk_hbm, zeros_hbm, out_hbm, pk_v, hist, rowidx, accum):
    c = lax.axis_index("c")
    s = lax.axis_index("s")
    w = s * _NC + c
    pltpu.sync_copy(pk_hbm.at[w], pk_v)

    zeros16 = jnp.zeros((16,), jnp.float32)

    def zbody(i, carry):
        hist[i >> 3, pl.ds((i & 7) * 16, 16)] = zeros16
        return carry
    lax.fori_loop(0, _DHR * 8, zbody, 0)

    iota16 = lax.broadcasted_iota(jnp.int32, (16,), 0)
    for i in range(_DHR // 16):
        rowidx[pl.ds(i * 16, 16)] = iota16 + i * 16

    ones = jnp.ones((16,), jnp.float32)

    def cbody(i, carry):
        idx = pk_v[i // 5, pl.ds((i % 5) * 16, 16)] >> 16
        plsc.addupdate_scatter(hist, [idx >> 7, idx & 127], ones)
        return carry
    lax.fori_loop(0, _EPW // 16, cbody, 0)

    @pl.when(s < 10)
    def _():
        pltpu.sync_copy(zeros_hbm.at[pl.ds(8 * s, 8)],
                        accum.at[pl.ds(8 * s, 8)])
    plsc.subcore_barrier()
    pltpu.sync_copy(hist, accum.at[rowidx], add=True)
    plsc.subcore_barrier()

    @pl.when(s < 10)
    def _():
        pltpu.sync_copy(accum.at[pl.ds(8 * s, 8)],
                        out_hbm.at[c, pl.ds(8 * s, 8)])


_NBUF = 3


@functools.cache
def _get_mp_kernel():
    return functools.partial(
        pl.kernel,
        out_type=jax.ShapeDtypeStruct((_NC, _NPAD, _D), jnp.float32),
        mesh=_mesh(),
        scratch_types=[
            pltpu.VMEM_SHARED((_NPAD, _D), jnp.float32),
        ]
        + [pltpu.VMEM((_CHUNK, _D), jnp.float32) for _ in range(_NBUF)]
        + [pltpu.VMEM((_CHUNK,), jnp.int32) for _ in range(3 * _NBUF)]
        + [pltpu.SemaphoreType.DMA for _ in range(3 * _NBUF)],
        compiler_params=pltpu.CompilerParams(needs_layout_passes=False),
    )(_mp_body)


def _mp_body(g_hbm, pk_hbm, zeros_hbm, out_hbm,
             accum, *bufs_and_sems):
    bufs = bufs_and_sems[:_NBUF]
    pkbufs = bufs_and_sems[_NBUF:2 * _NBUF]
    srcbufs = bufs_and_sems[2 * _NBUF:3 * _NBUF]
    dstbufs = bufs_and_sems[3 * _NBUF:4 * _NBUF]
    gsems = bufs_and_sems[4 * _NBUF:5 * _NBUF]
    ssems = bufs_and_sems[5 * _NBUF:6 * _NBUF]
    psems = bufs_and_sems[6 * _NBUF:]
    c = lax.axis_index("c")
    s = lax.axis_index("s")
    w = s * _NC + c

    def pkstart(j, b):
        pltpu.async_copy(pk_hbm.at[w, j], pkbufs[b], psems[b])

    def pkwait(b):
        pltpu.make_async_copy(pk_hbm.at[w, 0], pkbufs[b], psems[b]).wait()

    def unpack(b):
        for i in range(_CHUNK // 16):
            p = pkbufs[b][pl.ds(i * 16, 16)]
            srcbufs[b][pl.ds(i * 16, 16)] = p & 0xFFFF
            dstbufs[b][pl.ds(i * 16, 16)] = p >> 16

    def gstart(b):
        pltpu.async_copy(g_hbm.at[srcbufs[b]], bufs[b], gsems[b])

    def gwait(b):
        pltpu.make_async_copy(g_hbm.at[srcbufs[b]], bufs[b],
                              gsems[b]).wait()

    def sstart(b):
        pltpu.async_copy(bufs[b], accum.at[dstbufs[b]], ssems[b], add=True)

    def swait(b):
        pltpu.make_async_copy(bufs[b], accum.at[dstbufs[b]],
                              ssems[b]).wait()

    for b in range(_NBUF):
        pkstart(b, b)
    for b in range(_NBUF):
        pkwait(b)
        unpack(b)
        gstart(b)
        pkstart(b + _NBUF, b)
    pltpu.sync_copy(zeros_hbm.at[pl.ds(s * _ROWS_T, _ROWS_T)],
                    accum.at[pl.ds(s * _ROWS_T, _ROWS_T)])
    plsc.subcore_barrier()

    def body(k, carry):
        for b in range(_NBUF):
            gwait(b)
            sstart(b)
        for b in range(_NBUF):
            jn = k * _NBUF + b + _NBUF

            @pl.when(jn < _NCHUNK)
            def _():
                swait(b)
                pkwait(b)
                unpack(b)
                gstart(b)

                @pl.when(jn + _NBUF < _NCHUNK)
                def _():
                    pkstart(jn + _NBUF, b)
        return carry
    lax.fori_loop(0, _NCHUNK // _NBUF, body, 0)

    for b in range(_NCHUNK % _NBUF):
        gwait(b)
        sstart(b)
    for b in range(_NBUF):
        swait(b)

    plsc.subcore_barrier()
    pltpu.sync_copy(accum.at[pl.ds(s * _ROWS_T, _ROWS_T)],
                    out_hbm.at[c, pl.ds(s * _ROWS_T, _ROWS_T)])


def _elu(x):
    return jnp.where(x > 0, x, jnp.exp(jnp.where(x > 0, 0.0, x)) - 1.0)


def _row_spec(width):
    return pl.BlockSpec((_BLK, width), lambda i: (i, 0))


def _full_spec(shape):
    return pl.BlockSpec(shape, lambda i: tuple(0 for _ in shape))


def _t_spec():
    return pl.BlockSpec((_NC, _BLK, _D), lambda i: (0, i, 0))


def _tc0_body(x_ref, w0_ref, d0_ref, d1_ref, qe_ref, f0w_ref, f0b_ref,
              g0_ref, dinv_ref, q_ref):
    dinv = lax.rsqrt(d0_ref[...] + d1_ref[...] + 1.0)
    dinv_ref[...] = dinv
    g0_ref[...] = dinv * jnp.dot(x_ref[...], w0_ref[...],
                                 preferred_element_type=jnp.float32)

    @pl.when(pl.program_id(0) == 0)
    def _():
        q = jnp.dot(qe_ref[...], f0w_ref[...],
                    preferred_element_type=jnp.float32) + f0b_ref[...]
        q_ref[...] = _elu(q)


def _tc0(x, W0, d0, d1, qe, f0w, f0b):
    return pl.pallas_call(
        _tc0_body,
        grid=(_GRID,),
        in_specs=[_row_spec(_D), _full_spec((_D, _D)), _row_spec(1),
                  _row_spec(1), _full_spec((64, 256)), _full_spec((256, 64)),
                  _full_spec((1, 64))],
        out_specs=[_row_spec(_D), _row_spec(1), _full_spec((64, 64))],
        out_shape=[jax.ShapeDtypeStruct((_N, _D), jnp.float32),
                   jax.ShapeDtypeStruct((_N, 1), jnp.float32),
                   jax.ShapeDtypeStruct((64, 64), jnp.float32)],
    )(x, W0, d0, d1, qe, f0w, f0b)


def _tc_mid_body(emit_h, t_ref, g_ref, dinv_ref, b_ref, w_ref, *out_refs):
    dinv = dinv_ref[...]
    h = _elu(dinv * (t_ref[0] + t_ref[1] + g_ref[...]) + b_ref[...])
    if emit_h:
        out_refs[1][...] = h
    out_refs[0][...] = dinv * jnp.dot(h, w_ref[...],
                                      preferred_element_type=jnp.float32)


def _tc_mid(t, g, dinv, b, W_next, emit_h):
    out_specs = [_row_spec(_D)]
    out_shape = [jax.ShapeDtypeStruct((_N, _D), jnp.float32)]
    if emit_h:
        out_specs.append(_row_spec(_D))
        out_shape.append(jax.ShapeDtypeStruct((_N, _D), jnp.float32))
    return pl.pallas_call(
        functools.partial(_tc_mid_body, emit_h),
        grid=(_GRID,),
        in_specs=[_t_spec(), _row_spec(_D), _row_spec(1),
                  _full_spec((1, _D)), _full_spec((_D, _D))],
        out_specs=out_specs,
        out_shape=out_shape,
    )(t, g, dinv, b, W_next)


def _tc_final_body(t_ref, g_ref, dinv_ref, b_ref, r_ref, q_ref, batch_ref,
                   f1a_ref, f1b_ref, f1bias_ref, f2w_ref, f2b_ref, out_ref):
    dinv = dinv_ref[...]
    h = _elu(dinv * (t_ref[0] + t_ref[1] + g_ref[...]) + b_ref[...])
    h = h + r_ref[...]
    onehot = (batch_ref[...] ==
              lax.broadcasted_iota(jnp.int32, (_BLK, 64), 1)
              ).astype(jnp.float32)
    qx = jnp.dot(onehot, q_ref[...], preferred_element_type=jnp.float32)
    pre = (jnp.dot(h, f1a_ref[...], preferred_element_type=jnp.float32)
           + jnp.dot(qx, f1b_ref[...], preferred_element_type=jnp.float32)
           + f1bias_ref[...])
    h2 = _elu(pre)
    out_ref[...] = (jnp.dot(h2, f2w_ref[...],
                            preferred_element_type=jnp.float32)
                    + f2b_ref[...])


def _tc_final(t, g, dinv, b, r, q, batch2, f1a, f1b, f1bias, f2w, f2b):
    return pl.pallas_call(
        _tc_final_body,
        grid=(_GRID,),
        in_specs=[_t_spec(), _row_spec(_D), _row_spec(1),
                  _full_spec((1, _D)), _row_spec(_D), _full_spec((64, 64)),
                  _row_spec(1), _full_spec((_D, _D)), _full_spec((64, _D)),
                  _full_spec((1, _D)), _full_spec((_D, _D)),
                  _full_spec((1, _D))],
        out_specs=_row_spec(_D),
        out_shape=jax.ShapeDtypeStruct((_N, _D), jnp.float32),
    )(t, g, dinv, b, r, q, batch2, f1a, f1b, f1bias, f2w, f2b)


def kernel(x, edge_index, batch, question_embedding,
           W0, b0, W1, b1, W2, b2, W3, b3,
           fc0_W, fc0_b, fc1_W, fc1_b, fc2_W, fc2_b):
    src = edge_index[0].reshape(_NW, _NCHUNK, _CHUNK)
    dst = edge_index[1].reshape(_NW, _NCHUNK, _CHUNK)
    packed = src | (dst << 16)
    zeros_nd = jnp.zeros((_NPAD, _D), jnp.float32)

    degp = _get_deg_kernel()(packed, zeros_nd).reshape(_NC, _DHR * _D)
    d0 = degp[0, :_N].reshape(_N, 1)
    d1 = degp[1, :_N].reshape(_N, 1)

    g0, dinv, q = _tc0(x, W0, d0, d1, question_embedding, fc0_W,
                       fc0_b.reshape(1, 64))

    t = _get_mp_kernel()(g0, packed, zeros_nd)
    (g1,) = _tc_mid(t, g0, dinv, b0.reshape(1, _D), W1, False)
    t = _get_mp_kernel()(g1, packed, zeros_nd)
    g2, r = _tc_mid(t, g1, dinv, b1.reshape(1, _D), W2, True)
    t = _get_mp_kernel()(g2, packed, zeros_nd)
    (g3,) = _tc_mid(t, g2, dinv, b2.reshape(1, _D), W3, False)
    t = _get_mp_kernel()(g3, packed, zeros_nd)

    out = _tc_final(t, g3, dinv, b3.reshape(1, _D), r, q,
                    batch.reshape(_N, 1),
                    fc1_W[:_D], fc1_W[_D:], fc1_b.reshape(1, _D),
                    fc2_W, fc2_b.reshape(1, _D))
    return out
